# baseline (speedup 1.0000x reference)
import jax
import jax.numpy as jnp
from jax.experimental import pallas as pl
from jax.experimental.pallas import tpu as pltpu

_F32 = jnp.float32


def _lenet_kernel(x_ref, w1_ref, b1_ref, w2_ref, b2_ref,
                  wf1_ref, bf1_ref, wf2_ref, bf2_ref, wf3_ref, bf3_ref,
                  out_ref, xt_ref, m1_ref, p1_ref, m2_ref, xf_ref):
    bt = out_ref.shape[-1]

    xt_ref[...] = x_ref[...].reshape(784, bt)

    w1 = w1_ref[...]
    b1 = b1_ref[...]
    for y in range(26):
        z = jnp.dot(w1, xt_ref[pl.ds(28 * y, 84), :], preferred_element_type=_F32)
        z = jnp.maximum(z + b1, 0.0)
        ze = jnp.concatenate([z[16 * j: 16 * j + 8] for j in range(13)], axis=0)
        zo = jnp.concatenate([z[16 * j + 8: 16 * j + 16] for j in range(13)], axis=0)
        m1_ref[y] = jnp.maximum(ze, zo)

    for py in range(13):
        p1_ref[pl.ds(104 * py, 104), :] = jnp.maximum(m1_ref[2 * py], m1_ref[2 * py + 1])

    w2 = w2_ref[...]
    b2 = b2_ref[...]
    for y in range(11):
        z = jnp.dot(w2, p1_ref[pl.ds(104 * y, 312), :], preferred_element_type=_F32)
        z = jnp.maximum(z + b2, 0.0)
        ze = jnp.concatenate([z[32 * j: 32 * j + 16] for j in range(5)], axis=0)
        zo = jnp.concatenate([z[32 * j + 16: 32 * j + 32] for j in range(5)], axis=0)
        m2_ref[y] = jnp.maximum(ze, zo)

    for py in range(5):
        xf_ref[pl.ds(80 * py, 80), :] = jnp.maximum(m2_ref[2 * py], m2_ref[2 * py + 1])

    xf = xf_ref[...]
    z = jnp.maximum(jnp.dot(wf1_ref[...], xf, preferred_element_type=_F32) + bf1_ref[...], 0.0)
    z = jnp.maximum(jnp.dot(wf2_ref[...], z, preferred_element_type=_F32) + bf2_ref[...], 0.0)
    logits = jnp.dot(wf3_ref[...], z, preferred_element_type=_F32) + bf3_ref[...]

    m = jnp.max(logits, axis=0, keepdims=True)
    lse = jnp.log(jnp.sum(jnp.exp(logits - m), axis=0, keepdims=True)) + m
    out_ref[...] = (logits - lse).astype(out_ref.dtype)


def _conv1_matrix(w, b):
    base = jnp.pad(w[:, 0], ((0, 2), (0, 0), (0, 25))).reshape(8, 84)
    v = jnp.pad(base, ((0, 0), (0, 26)))
    rows = jnp.tile(v, (1, 26))[:, :26 * 109].reshape(8, 26, 109)
    mat = jnp.transpose(rows[:, :, :84], (1, 0, 2)).reshape(208, 84)
    bias = jnp.tile(jnp.pad(b, (0, 2)), 26).reshape(208, 1)
    return mat, bias


def _conv2_matrix(w, b):
    base = jnp.transpose(w, (0, 2, 3, 1))
    base = jnp.pad(base, ((0, 0), (0, 0), (0, 10), (0, 2)))
    v = jnp.pad(base.reshape(16, 312), ((0, 0), (0, 80)))
    rows = jnp.tile(v, (1, 11))[:, :11 * 384].reshape(16, 11, 384)
    mat = jnp.transpose(rows[:, :, :312], (1, 0, 2)).reshape(176, 312)
    bias = jnp.tile(b.reshape(1, 16), (11, 1)).reshape(176, 1)
    return mat, bias


def kernel(conv1_w, conv1_b, conv2_w, conv2_b, fc1_w, fc1_b,
           fc2_w, fc2_b, fc3_w, fc3_b, x, *, block_b=512):
    B = x.shape[0]
    bt = block_b
    nb = (B + bt - 1) // bt
    bp = nb * bt

    xT = jnp.transpose(x.astype(_F32), (1, 2, 3, 0))
    if bp != B:
        xT = jnp.pad(xT, ((0, 0), (0, 0), (0, 0), (0, bp - B)))

    w1r, b1r = _conv1_matrix(conv1_w, conv1_b)
    w2r, b2r = _conv2_matrix(conv2_w, conv2_b)
    wf1 = fc1_w.reshape(120, 16, 5, 5).transpose(0, 2, 3, 1).reshape(120, 400)
    bf1 = fc1_b.reshape(120, 1)
    bf2 = fc2_b.reshape(84, 1)
    bf3 = fc3_b.reshape(10, 1)

    flops_per_img = 2 * (26 * 208 * 84 + 11 * 176 * 312 + 400 * 120 + 120 * 84 + 84 * 10)
    cost = pl.CostEstimate(
        flops=flops_per_img * bp,
        transcendentals=11 * bp,
        bytes_accessed=(784 + 10) * 4 * bp,
    )

    out = pl.pallas_call(
        _lenet_kernel,
        out_shape=jax.ShapeDtypeStruct((10, bp), _F32),
        grid=(nb,),
        in_specs=[
            pl.BlockSpec((1, 28, 28, bt), lambda b: (0, 0, 0, b)),
            pl.BlockSpec((208, 84), lambda b: (0, 0)),
            pl.BlockSpec((208, 1), lambda b: (0, 0)),
            pl.BlockSpec((176, 312), lambda b: (0, 0)),
            pl.BlockSpec((176, 1), lambda b: (0, 0)),
            pl.BlockSpec((120, 400), lambda b: (0, 0)),
            pl.BlockSpec((120, 1), lambda b: (0, 0)),
            pl.BlockSpec((84, 120), lambda b: (0, 0)),
            pl.BlockSpec((84, 1), lambda b: (0, 0)),
            pl.BlockSpec((10, 84), lambda b: (0, 0)),
            pl.BlockSpec((10, 1), lambda b: (0, 0)),
        ],
        out_specs=pl.BlockSpec((10, bt), lambda b: (0, b)),
        scratch_shapes=[
            pltpu.VMEM((784, bt), _F32),
            pltpu.VMEM((26, 104, bt), _F32),
            pltpu.VMEM((1352, bt), _F32),
            pltpu.VMEM((11, 80, bt), _F32),
            pltpu.VMEM((400, bt), _F32),
        ],
        compiler_params=pltpu.CompilerParams(
            dimension_semantics=("parallel",),
            vmem_limit_bytes=48 * 1024 * 1024,
        ),
        cost_estimate=cost,
    )(xT, w1r, b1r, w2r, b2r, wf1, bf1, fc2_w, bf2, fc3_w, bf3)

    return jnp.transpose(out[:, :B], (1, 0))


# speedup vs baseline: 1.4537x; 1.4537x over previous
import jax
import jax.numpy as jnp
from jax.experimental import pallas as pl
from jax.experimental.pallas import tpu as pltpu

_F32 = jnp.float32


def _lenet_kernel(x_ref, w1_ref, b1_ref, w2_ref, b2_ref,
                  wf1_ref, bf1_ref, wf2_ref, bf2_ref, wf3_ref, bf3_ref,
                  out_ref, xt_ref, m1_ref, p1_ref, m2_ref, xf_ref):
    bt = out_ref.shape[-1]

    xt_ref[...] = x_ref[...].reshape(784, bt)

    w1 = w1_ref[...]
    b1 = b1_ref[...]
    for y in range(26):
        z = jnp.dot(w1, xt_ref[pl.ds(28 * y, 84), :], preferred_element_type=_F32)
        z = jnp.maximum(z + b1, 0.0)
        ze = jnp.concatenate([z[16 * j: 16 * j + 8] for j in range(13)], axis=0)
        zo = jnp.concatenate([z[16 * j + 8: 16 * j + 16] for j in range(13)], axis=0)
        m1_ref[y] = jnp.maximum(ze, zo)

    for py in range(13):
        p1_ref[pl.ds(104 * py, 104), :] = jnp.maximum(m1_ref[2 * py], m1_ref[2 * py + 1])

    w2 = w2_ref[...]
    b2 = b2_ref[...]
    for y in range(11):
        z = jnp.dot(w2, p1_ref[pl.ds(104 * y, 312), :], preferred_element_type=_F32)
        z = jnp.maximum(z + b2, 0.0)
        ze = jnp.concatenate([z[32 * j: 32 * j + 16] for j in range(5)], axis=0)
        zo = jnp.concatenate([z[32 * j + 16: 32 * j + 32] for j in range(5)], axis=0)
        m2_ref[y] = jnp.maximum(ze, zo)

    for py in range(5):
        xf_ref[pl.ds(80 * py, 80), :] = jnp.maximum(m2_ref[2 * py], m2_ref[2 * py + 1])

    xf = xf_ref[...]
    z = jnp.maximum(jnp.dot(wf1_ref[...], xf, preferred_element_type=_F32) + bf1_ref[...], 0.0)
    z = jnp.maximum(jnp.dot(wf2_ref[...], z, preferred_element_type=_F32) + bf2_ref[...], 0.0)
    logits = jnp.dot(wf3_ref[...], z, preferred_element_type=_F32) + bf3_ref[...]

    m = jnp.max(logits, axis=0, keepdims=True)
    lse = jnp.log(jnp.sum(jnp.exp(logits - m), axis=0, keepdims=True)) + m
    out_ref[...] = (logits - lse).astype(out_ref.dtype)


def _conv1_matrix(w, b):
    base = jnp.pad(w[:, 0], ((0, 2), (0, 0), (0, 25))).reshape(8, 84)
    v = jnp.pad(base, ((0, 0), (0, 26)))
    rows = jnp.tile(v, (1, 26))[:, :26 * 109].reshape(8, 26, 109)
    mat = jnp.transpose(rows[:, :, :84], (1, 0, 2)).reshape(208, 84)
    bias = jnp.tile(jnp.pad(b, (0, 2)), 26).reshape(208, 1)
    return mat, bias


def _conv2_matrix(w, b):
    base = jnp.transpose(w, (0, 2, 3, 1))
    base = jnp.pad(base, ((0, 0), (0, 0), (0, 10), (0, 2)))
    v = jnp.pad(base.reshape(16, 312), ((0, 0), (0, 80)))
    rows = jnp.tile(v, (1, 11))[:, :11 * 384].reshape(16, 11, 384)
    mat = jnp.transpose(rows[:, :, :312], (1, 0, 2)).reshape(176, 312)
    bias = jnp.tile(b.reshape(1, 16), (11, 1)).reshape(176, 1)
    return mat, bias


def kernel(conv1_w, conv1_b, conv2_w, conv2_b, fc1_w, fc1_b,
           fc2_w, fc2_b, fc3_w, fc3_b, x, *, block_b=512):
    B = x.shape[0]
    bt = block_b
    nb = (B + bt - 1) // bt
    bp = nb * bt

    xT = jnp.transpose(x.astype(_F32), (2, 3, 1, 0))
    if bp != B:
        xT = jnp.pad(xT, ((0, 0), (0, 0), (0, 0), (0, bp - B)))

    w1r, b1r = _conv1_matrix(conv1_w, conv1_b)
    w2r, b2r = _conv2_matrix(conv2_w, conv2_b)
    wf1 = fc1_w.reshape(120, 16, 5, 5).transpose(0, 2, 3, 1).reshape(120, 400)
    bf1 = fc1_b.reshape(120, 1)
    bf2 = fc2_b.reshape(84, 1)
    bf3 = fc3_b.reshape(10, 1)

    flops_per_img = 2 * (26 * 208 * 84 + 11 * 176 * 312 + 400 * 120 + 120 * 84 + 84 * 10)
    cost = pl.CostEstimate(
        flops=flops_per_img * bp,
        transcendentals=11 * bp,
        bytes_accessed=(784 + 10) * 4 * bp,
    )

    out = pl.pallas_call(
        _lenet_kernel,
        out_shape=jax.ShapeDtypeStruct((10, bp), _F32),
        grid=(nb,),
        in_specs=[
            pl.BlockSpec((28, 28, 1, bt), lambda b: (0, 0, 0, b)),
            pl.BlockSpec((208, 84), lambda b: (0, 0)),
            pl.BlockSpec((208, 1), lambda b: (0, 0)),
            pl.BlockSpec((176, 312), lambda b: (0, 0)),
            pl.BlockSpec((176, 1), lambda b: (0, 0)),
            pl.BlockSpec((120, 400), lambda b: (0, 0)),
            pl.BlockSpec((120, 1), lambda b: (0, 0)),
            pl.BlockSpec((84, 120), lambda b: (0, 0)),
            pl.BlockSpec((84, 1), lambda b: (0, 0)),
            pl.BlockSpec((10, 84), lambda b: (0, 0)),
            pl.BlockSpec((10, 1), lambda b: (0, 0)),
        ],
        out_specs=pl.BlockSpec((10, bt), lambda b: (0, b)),
        scratch_shapes=[
            pltpu.VMEM((784, bt), _F32),
            pltpu.VMEM((26, 104, bt), _F32),
            pltpu.VMEM((1352, bt), _F32),
            pltpu.VMEM((11, 80, bt), _F32),
            pltpu.VMEM((400, bt), _F32),
        ],
        compiler_params=pltpu.CompilerParams(
            dimension_semantics=("parallel",),
            vmem_limit_bytes=48 * 1024 * 1024,
        ),
        cost_estimate=cost,
    )(xT, w1r, b1r, w2r, b2r, wf1, bf1, fc2_w, bf2, fc3_w, bf3)

    return jnp.transpose(out[:, :B], (1, 0))
